# baseline (speedup 1.0000x reference)
import functools

import jax
import jax.numpy as jnp
from jax import lax
from jax.experimental import pallas as pl
from jax.experimental.pallas import tpu as pltpu
from jax.experimental.pallas import tpu_sc as plsc

Z_DIM = 101
LATENT_DIM = 92
N_ATOMS = 100000

L = 16
ROWS_PER_CHUNK = 625
N_CHUNKS = N_ATOMS // ROWS_PER_CHUNK
N_WORKERS = 32
CHUNKS_PER_WORKER = N_CHUNKS // N_WORKERS
ROWS_PER_WORKER = ROWS_PER_CHUNK * CHUNKS_PER_WORKER
IDX_PAD = 3136
N_COLG = 6
RU = 5
TAB_PAD = Z_DIM * LATENT_DIM + 20


def _make_sc_gather():
    mesh = plsc.VectorSubcoreMesh(core_axis_name="c", subcore_axis_name="s")
    nc = mesh.num_cores

    @functools.partial(
        pl.kernel,
        out_type=jax.ShapeDtypeStruct((N_CHUNKS, ROWS_PER_CHUNK, LATENT_DIM),
                                      jnp.float32),
        mesh=mesh,
        scratch_types=[
            pltpu.VMEM((TAB_PAD,), jnp.float32),
            pltpu.VMEM((1, IDX_PAD), jnp.int32),
            pltpu.VMEM((ROWS_PER_CHUNK, LATENT_DIM), jnp.float32),
        ],
        compiler_params=pltpu.CompilerParams(use_tc_tiling_on_sc=False,
                                             needs_layout_passes=False),
    )
    def gather_kernel(z_hbm, tab_hbm, out_hbm, tab_v, idx_v, stag_v):
        wid = lax.axis_index("s") * nc + lax.axis_index("c")
        base = wid * CHUNKS_PER_WORKER
        pltpu.sync_copy(tab_hbm, tab_v)
        pltpu.sync_copy(z_hbm.at[wid], idx_v)

        lanes = lax.iota(jnp.int32, L)
        zeros = jnp.zeros((L,), jnp.int32)
        tail_mask = lanes < LATENT_DIM - (N_COLG - 1) * L
        colvecs = [g * L + lanes for g in range(N_COLG)]

        def chunk(c, carry):
            @plsc.parallel_loop(0, ROWS_PER_CHUNK, step=RU, unroll=5)
            def rowblk(r0):
                vidx = idx_v[0, pl.ds(c * ROWS_PER_CHUNK + r0, L)]
                for u in range(RU):
                    r = r0 + u
                    s = vidx[u] * LATENT_DIM
                    for g in range(N_COLG - 1):
                        stag_v[r, pl.ds(g * L, L)] = \
                            tab_v[pl.ds(s + g * L, L)]
                    vals = tab_v[pl.ds(s + (N_COLG - 1) * L, L)]
                    plsc.store_scatter(stag_v, [zeros + r, colvecs[-1]],
                                       vals, mask=tail_mask)

            pltpu.sync_copy(stag_v, out_hbm.at[base + c])
            return carry

        lax.fori_loop(0, CHUNKS_PER_WORKER, chunk, 0)

    return gather_kernel


_sc_gather = _make_sc_gather()


@jax.jit
def kernel(z, features):
    zw = z.reshape(N_WORKERS, ROWS_PER_WORKER)
    zw = jnp.pad(zw, ((0, 0), (0, IDX_PAD - ROWS_PER_WORKER)))
    z3 = zw.reshape(N_WORKERS, 1, IDX_PAD)
    tab = jnp.pad(features.reshape(Z_DIM * LATENT_DIM),
                  (0, TAB_PAD - Z_DIM * LATENT_DIM))
    out = _sc_gather(z3, tab)
    return out.reshape(N_ATOMS, LATENT_DIM)


# speedup vs baseline: 1.2272x; 1.2272x over previous
import functools

import jax
import jax.numpy as jnp
from jax import lax
from jax.experimental import pallas as pl
from jax.experimental.pallas import tpu as pltpu
from jax.experimental.pallas import tpu_sc as plsc

Z_DIM = 101
LATENT_DIM = 92
N_ATOMS = 100000

L = 16
ROWS_PER_CHUNK = 125
N_CHUNKS = N_ATOMS // ROWS_PER_CHUNK
N_WORKERS = 32
CHUNKS_PER_WORKER = N_CHUNKS // N_WORKERS
ROWS_PER_WORKER = ROWS_PER_CHUNK * CHUNKS_PER_WORKER
IDX_PAD = 3136
N_COLG = 6
RU = 5
TAB_PAD = Z_DIM * LATENT_DIM + 20


def _make_sc_gather():
    mesh = plsc.VectorSubcoreMesh(core_axis_name="c", subcore_axis_name="s")
    nc = mesh.num_cores

    @functools.partial(
        pl.kernel,
        out_type=jax.ShapeDtypeStruct((N_CHUNKS, ROWS_PER_CHUNK, LATENT_DIM),
                                      jnp.float32),
        mesh=mesh,
        scratch_types=[
            pltpu.VMEM((TAB_PAD,), jnp.float32),
            pltpu.VMEM((1, IDX_PAD), jnp.int32),
            pltpu.VMEM((ROWS_PER_CHUNK, LATENT_DIM), jnp.float32),
            pltpu.VMEM((ROWS_PER_CHUNK, LATENT_DIM), jnp.float32),
            pltpu.SemaphoreType.DMA,
        ],
        compiler_params=pltpu.CompilerParams(use_tc_tiling_on_sc=False,
                                             needs_layout_passes=False),
    )
    def gather_kernel(z_hbm, tab_hbm, out_hbm, tab_v, idx_v,
                      stag0, stag1, sem):
        wid = lax.axis_index("s") * nc + lax.axis_index("c")
        base = wid * CHUNKS_PER_WORKER
        pltpu.sync_copy(tab_hbm, tab_v)
        pltpu.sync_copy(z_hbm.at[wid], idx_v)

        lanes = lax.iota(jnp.int32, L)
        zeros = jnp.zeros((L,), jnp.int32)
        tail_mask = lanes < LATENT_DIM - (N_COLG - 1) * L
        tail_cols = (N_COLG - 1) * L + lanes

        def fill(c, stag_v):
            @plsc.parallel_loop(0, ROWS_PER_CHUNK, step=RU, unroll=5)
            def rowblk(r0):
                vidx = idx_v[0, pl.ds(c * ROWS_PER_CHUNK + r0, L)]
                for u in range(RU):
                    r = r0 + u
                    s = vidx[u] * LATENT_DIM
                    for g in range(N_COLG - 1):
                        stag_v[r, pl.ds(g * L, L)] = \
                            tab_v[pl.ds(s + g * L, L)]
                    vals = tab_v[pl.ds(s + (N_COLG - 1) * L, L)]
                    plsc.store_scatter(stag_v, [zeros + r, tail_cols],
                                       vals, mask=tail_mask)

        def chunk(c, carry):
            def phase(stag_v):
                @pl.when(c >= 2)
                def _():
                    pltpu.make_async_copy(stag_v, out_hbm.at[base + c - 2],
                                          sem).wait()
                fill(c, stag_v)
                pltpu.async_copy(stag_v, out_hbm.at[base + c], sem)

            lax.cond(c % 2 == 0, lambda: phase(stag0), lambda: phase(stag1))
            return carry

        lax.fori_loop(0, CHUNKS_PER_WORKER, chunk, 0)
        last = base + CHUNKS_PER_WORKER - 1
        pltpu.make_async_copy(stag1, out_hbm.at[last - 1], sem).wait()
        pltpu.make_async_copy(stag0, out_hbm.at[last], sem).wait()

    return gather_kernel


_sc_gather = _make_sc_gather()


@jax.jit
def kernel(z, features):
    zw = z.reshape(N_WORKERS, ROWS_PER_WORKER)
    zw = jnp.pad(zw, ((0, 0), (0, IDX_PAD - ROWS_PER_WORKER)))
    z3 = zw.reshape(N_WORKERS, 1, IDX_PAD)
    tab = jnp.pad(features.reshape(Z_DIM * LATENT_DIM),
                  (0, TAB_PAD - Z_DIM * LATENT_DIM))
    out = _sc_gather(z3, tab)
    return out.reshape(N_ATOMS, LATENT_DIM)
